# baseline (speedup 1.0000x reference)
import functools

import jax
import jax.numpy as jnp
from jax import lax
from jax.experimental import pallas as pl
from jax.experimental.pallas import tpu as pltpu
from jax.experimental.pallas import tpu_sc as plsc

_TR = 1024


def _pass_a_body(hw, xT_ref, x_ref, mrow_ref, mcol_ref,
                 qn_ref, fsn_ref, w_ref, idx_ref, fore_ref,
                 fs_cn_ref):
    t = pl.program_id(0)

    @pl.when(t == 0)
    def _():
        xb = x_ref[...]
        mr = mrow_ref[...]
        fs = xb * (1.0 - mr)
        nrm = jnp.sqrt(jnp.sum(fs * fs, axis=0, keepdims=True)) + 1e-8
        fs_cn_ref[...] = fs / nrm

    xt = xT_ref[...]
    mc = mcol_ref[...]

    q = xt * mc
    qn = q / (jnp.sqrt(jnp.sum(q * q, axis=1, keepdims=True)) + 1e-8)
    qn_ref[...] = qn

    s_rows = xt * (1.0 - mc)
    sn = s_rows / (jnp.sqrt(jnp.sum(s_rows * s_rows, axis=1, keepdims=True))
                   + 1e-8)
    fsn_ref[...] = sn

    simi = jnp.dot(qn, fs_cn_ref[...], preferred_element_type=jnp.float32)
    w = jnp.max(simi, axis=1, keepdims=True)
    w_ref[...] = w
    iif = lax.broadcasted_iota(jnp.int32, simi.shape, 1).astype(jnp.float32)
    idxf = jnp.min(jnp.where(simi == w, iif, float(hw)), axis=1,
                   keepdims=True)
    idx_ref[...] = idxf.astype(jnp.int32)

    c0 = jnp.max(simi[:, 0:1], axis=0, keepdims=True)

    @pl.when(t == 0)
    def _():
        fore_ref[...] = c0

    @pl.when(t != 0)
    def _():
        fore_ref[...] = jnp.maximum(fore_ref[...], c0)


def _run_pass_a(xTb, xrb, mrowb, mcolb):
    HW, C = xTb.shape
    T = HW // _TR
    f32 = jnp.float32
    return pl.pallas_call(
        functools.partial(_pass_a_body, HW),
        grid=(T,),
        in_specs=[
            pl.BlockSpec((_TR, C), lambda t: (t, 0)),
            pl.BlockSpec((C, HW), lambda t: (0, 0)),
            pl.BlockSpec((1, HW), lambda t: (0, 0)),
            pl.BlockSpec((_TR, 1), lambda t: (t, 0)),
        ],
        out_specs=[
            pl.BlockSpec((_TR, C), lambda t: (t, 0)),
            pl.BlockSpec((_TR, C), lambda t: (t, 0)),
            pl.BlockSpec((_TR, 1), lambda t: (t, 0)),
            pl.BlockSpec((_TR, 1), lambda t: (t, 0)),
            pl.BlockSpec((1, 1), lambda t: (0, 0)),
        ],
        out_shape=[
            jax.ShapeDtypeStruct((HW, C), f32),
            jax.ShapeDtypeStruct((HW, C), f32),
            jax.ShapeDtypeStruct((HW, 1), f32),
            jax.ShapeDtypeStruct((HW, 1), jnp.int32),
            jax.ShapeDtypeStruct((1, 1), f32),
        ],
        scratch_shapes=[pltpu.VMEM((C, HW), f32)],
    )(xTb, xrb, mrowb, mcolb)


def _run_pass_b(fsn, idx2, want_att):
    hw, C = fsn.shape
    info = plsc.get_sparse_core_info()
    NC, NS, L = info.num_cores, info.num_subcores, info.num_lanes
    NW = NC * NS
    RPW = hw // NW
    SCH = hw // 128 // NS
    f32 = jnp.float32

    mesh = plsc.VectorSubcoreMesh(core_axis_name="c", subcore_axis_name="s")

    out_type = [jax.ShapeDtypeStruct((hw, C), f32)]
    scratch = [
        pltpu.VMEM((1, 128), jnp.int32),
        pltpu.VMEM((RPW, C), f32),
        pltpu.SemaphoreType.DMA,
    ]
    if want_att:
        out_type.append(jax.ShapeDtypeStruct((hw, 16), f32))
        scratch += [
            pltpu.VMEM((SCH, 128), jnp.int32),
            pltpu.VMEM((128, 16), f32),
            pltpu.VMEM_SHARED((hw, 16), f32),
        ]

    def body(fsn_hbm, idx2_hbm, sel_hbm, idx_v, rows_v, sem,
             ones_hbm=None, zeros_hbm=None, att_hbm=None,
             sidx_v=None, stage_v=None, attsh=None):
        cid = lax.axis_index("c")
        sid = lax.axis_index("s")
        wid = cid * NS + sid
        base = wid * RPW
        pltpu.sync_copy(idx2_hbm.at[pl.ds(wid * (RPW // 128), RPW // 128)],
                        idx_v)

        if want_att:
            @pl.when(cid == NC - 1)
            def _():
                pltpu.sync_copy(zeros_hbm, stage_v)
                for k in range(hw // (NS * 128)):
                    pltpu.sync_copy(
                        stage_v,
                        attsh.at[pl.ds(sid * (hw // NS) + k * 128, 128)])
                pltpu.sync_copy(idx2_hbm.at[pl.ds(sid * SCH, SCH)], sidx_v)

            plsc.subcore_barrier()

            @pl.when(cid == NC - 1)
            def _():
                pltpu.sync_copy(ones_hbm, stage_v)
                for j in range(SCH):
                    pltpu.sync_copy(stage_v, attsh.at[sidx_v.at[j]],
                                    add=True)

        copies = [
            pltpu.async_copy(fsn_hbm.at[idx_v.at[j]],
                             rows_v.at[pl.ds(j * 128, 128)], sem)
            for j in range(RPW // 128)
        ]
        for cp in copies:
            cp.wait()
        pltpu.sync_copy(rows_v, sel_hbm.at[pl.ds(base, RPW)])

        if want_att:
            plsc.subcore_barrier()

            @pl.when((sid == 0) & (cid == NC - 1))
            def _():
                pltpu.sync_copy(attsh, att_hbm)

    if want_att:
        def entry(fsn_hbm, idx2_hbm, ones_hbm, zeros_hbm, sel_hbm, att_hbm,
                  idx_v, rows_v, sem, sidx_v, stage_v, attsh):
            body(fsn_hbm, idx2_hbm, sel_hbm, idx_v, rows_v, sem,
                 ones_hbm=ones_hbm, zeros_hbm=zeros_hbm, att_hbm=att_hbm,
                 sidx_v=sidx_v, stage_v=stage_v, attsh=attsh)
    else:
        def entry(fsn_hbm, idx2_hbm, sel_hbm, idx_v, rows_v, sem):
            body(fsn_hbm, idx2_hbm, sel_hbm, idx_v, rows_v, sem)

    kern = pl.kernel(
        entry,
        out_type=out_type,
        mesh=mesh,
        scratch_types=scratch,
        compiler_params=pltpu.CompilerParams(use_tc_tiling_on_sc=False),
    )
    if want_att:
        ones = jnp.ones((128, 16), jnp.float32)
        zeros = jnp.zeros((128, 16), jnp.float32)
        return kern(fsn, idx2, ones, zeros)
    return kern(fsn, idx2)


def _pass_c_body(C, has_att, *refs):
    if has_att:
        (qn_ref, sel_ref, w_ref, mcol_ref, xT_ref, fore_ref,
         wct_ref, bc_ref, attp_ref, out_ref, att_ref) = refs
        att_ref[...] = jnp.minimum(attp_ref[:, 0:1], 1.0)
    else:
        (qn_ref, sel_ref, w_ref, mcol_ref, xT_ref, fore_ref,
         wct_ref, bc_ref, out_ref) = refs
    w = w_ref[...]
    mx = jnp.max(w)
    e = jnp.exp(w - mx)
    sm = e / jnp.sum(e)

    sel = sel_ref[...]
    qn = qn_ref[...]
    hyb = (jnp.dot(sel, wct_ref[:C, :], preferred_element_type=jnp.float32)
           * sm
           + jnp.dot(qn, wct_ref[C:, :], preferred_element_type=jnp.float32)
           + bc_ref[...])
    vm = jnp.where(fore_ref[...] > 0.5, mcol_ref[0:1, :], 0.0)
    refined = hyb * vm + qn * (1.0 - vm)
    mc = mcol_ref[...]
    out_ref[...] = refined * mc + xT_ref[...] * (1.0 - mc)


def _run_pass_c(qn, sel, w, mcol, xT, fore, wcT, bc2, attp=None):
    HW, C = qn.shape
    f32 = jnp.float32
    has_att = attp is not None
    in_specs = [
        pl.BlockSpec((HW, C), lambda: (0, 0)),
        pl.BlockSpec((HW, C), lambda: (0, 0)),
        pl.BlockSpec((HW, 1), lambda: (0, 0)),
        pl.BlockSpec((HW, 1), lambda: (0, 0)),
        pl.BlockSpec((HW, C), lambda: (0, 0)),
        pl.BlockSpec((1, 1), lambda: (0, 0)),
        pl.BlockSpec((2 * C, C), lambda: (0, 0)),
        pl.BlockSpec((1, C), lambda: (0, 0)),
    ]
    out_specs = [pl.BlockSpec((HW, C), lambda: (0, 0))]
    out_shape = [jax.ShapeDtypeStruct((HW, C), f32)]
    args = [qn, sel, w, mcol, xT, fore, wcT, bc2]
    if has_att:
        in_specs.append(pl.BlockSpec((HW, 16), lambda: (0, 0)))
        out_specs.append(pl.BlockSpec((HW, 1), lambda: (0, 0)))
        out_shape.append(jax.ShapeDtypeStruct((HW, 1), f32))
        args.append(attp)
    return pl.pallas_call(
        functools.partial(_pass_c_body, C, has_att),
        grid=(),
        in_specs=in_specs,
        out_specs=out_specs,
        out_shape=out_shape,
    )(*args)


def kernel(x, mask, Wc, bc):
    B, C, H, Wd = x.shape
    HW = H * Wd
    xr = x.reshape(B, C, HW)
    xT = xr.transpose(0, 2, 1)
    mflat = mask.reshape(B, HW)
    wcT = Wc.T.reshape(2 * C, C)
    bc2 = bc.reshape(1, C)

    outs = []
    attv = None
    for b in range(B):
        mrow = mflat[b].reshape(1, HW)
        mcol = mflat[b].reshape(HW, 1)
        qn, fsn, w, idx, fore = _run_pass_a(xT[b], xr[b], mrow, mcol)
        idx2 = idx.reshape(HW // 128, 128)
        last = b == B - 1
        if last:
            sel, attp = _run_pass_b(fsn, idx2, True)
            outT, attv = _run_pass_c(qn, sel, w, mcol, xT[b], fore,
                                     wcT, bc2, attp)
        else:
            (sel,) = _run_pass_b(fsn, idx2, False)
            (outT,) = _run_pass_c(qn, sel, w, mcol, xT[b], fore, wcT, bc2)
        outs.append(outT)

    out = jnp.stack(outs, 0).transpose(0, 2, 1).reshape(B, C, H, Wd)
    att = jnp.broadcast_to(attv.reshape(1, HW), (B, HW)).reshape(B, 1, H, Wd)
    att = jnp.repeat(jnp.repeat(att, 8, axis=2), 8, axis=3)
    return out, att


# speedup vs baseline: 1.0889x; 1.0889x over previous
import functools

import jax
import jax.numpy as jnp
from jax import lax
from jax.experimental import pallas as pl
from jax.experimental.pallas import tpu as pltpu
from jax.experimental.pallas import tpu_sc as plsc

_TR = 1024


def _pass_a_body(hw, nb, xT_ref, x_ref, mrow_ref, mcol_ref,
                 qn_ref, fsn_ref, w_ref, idx_ref, fore_ref,
                 fs_cn_ref):
    t = pl.program_id(1)

    @pl.when(t == 0)
    def _():
        xb = x_ref[0]
        mr = mrow_ref[0]
        fs = xb * (1.0 - mr)
        nrm = jnp.sqrt(jnp.sum(fs * fs, axis=0, keepdims=True)) + 1e-8
        fs_cn_ref[...] = fs / nrm

    xt = xT_ref[0]
    mc = mcol_ref[0]

    q = xt * mc
    qn = q / (jnp.sqrt(jnp.sum(q * q, axis=1, keepdims=True)) + 1e-8)
    qn_ref[0] = qn

    s_rows = xt * (1.0 - mc)
    sn = s_rows / (jnp.sqrt(jnp.sum(s_rows * s_rows, axis=1, keepdims=True))
                   + 1e-8)
    fsn_ref[0] = sn

    simi = jnp.dot(qn, fs_cn_ref[...], preferred_element_type=jnp.float32)
    w = jnp.max(simi, axis=1, keepdims=True)
    w_ref[0] = w
    iif = lax.broadcasted_iota(jnp.int32, simi.shape, 1).astype(jnp.float32)
    idxf = jnp.min(jnp.where(simi == w, iif, float(hw)), axis=1,
                   keepdims=True)
    idx = idxf.astype(jnp.int32)
    idx_ref[0] = idx

    c0 = jnp.max(simi[:, 0:1], axis=0, keepdims=True)

    @pl.when(t == 0)
    def _():
        fore_ref[0] = c0

    @pl.when(t != 0)
    def _():
        fore_ref[0] = jnp.maximum(fore_ref[0], c0)


def _run_pass_a(xT, xr, mrow, mcol):
    B, HW, C = xT.shape
    T = HW // _TR
    f32 = jnp.float32
    return pl.pallas_call(
        functools.partial(_pass_a_body, HW, B),
        grid=(B, T),
        in_specs=[
            pl.BlockSpec((1, _TR, C), lambda b, t: (b, t, 0)),
            pl.BlockSpec((1, C, HW), lambda b, t: (b, 0, 0)),
            pl.BlockSpec((1, 1, HW), lambda b, t: (b, 0, 0)),
            pl.BlockSpec((1, _TR, 1), lambda b, t: (b, t, 0)),
        ],
        out_specs=[
            pl.BlockSpec((1, _TR, C), lambda b, t: (b, t, 0)),
            pl.BlockSpec((1, _TR, C), lambda b, t: (b, t, 0)),
            pl.BlockSpec((1, _TR, 1), lambda b, t: (b, t, 0)),
            pl.BlockSpec((1, _TR, 1), lambda b, t: (b, t, 0)),
            pl.BlockSpec((1, 1, 1), lambda b, t: (b, 0, 0)),
        ],
        out_shape=[
            jax.ShapeDtypeStruct((B, HW, C), f32),
            jax.ShapeDtypeStruct((B, HW, C), f32),
            jax.ShapeDtypeStruct((B, HW, 1), f32),
            jax.ShapeDtypeStruct((B, HW, 1), jnp.int32),
            jax.ShapeDtypeStruct((B, 1, 1), f32),
        ],
        scratch_shapes=[pltpu.VMEM((C, HW), f32)],
    )(xT, xr, mrow, mcol)


def _run_pass_b(fsn_flat, idx2, hw):
    ROWS, C = fsn_flat.shape
    info = plsc.get_sparse_core_info()
    NC, NS, L = info.num_cores, info.num_subcores, info.num_lanes
    NW = NC * NS
    RPW = ROWS // NW
    NCHUNK = RPW // 128
    f32 = jnp.float32

    mesh = plsc.VectorSubcoreMesh(core_axis_name="c", subcore_axis_name="s")

    @functools.partial(
        pl.kernel,
        out_type=[
            jax.ShapeDtypeStruct((ROWS, C), f32),
            jax.ShapeDtypeStruct((hw, 16), f32),
        ],
        mesh=mesh,
        scratch_types=[
            pltpu.VMEM((NCHUNK, 128), jnp.int32),
            pltpu.VMEM((RPW, C), f32),
            pltpu.SemaphoreType.DMA,
            pltpu.SemaphoreType.DMA,
            pltpu.VMEM((128, 16), f32),
            pltpu.VMEM_SHARED((hw, 16), f32),
        ],
        compiler_params=pltpu.CompilerParams(use_tc_tiling_on_sc=False),
    )
    def sc_kernel(fsn_hbm, idx2_hbm, ones_hbm, zeros_hbm,
                  sel_hbm, att_hbm,
                  idx_v, rows_v, sem, sem2, stage_v, attsh):
        cid = lax.axis_index("c")
        sid = lax.axis_index("s")
        wid = cid * NS + sid
        base = wid * RPW
        rowblk = wid * NCHUNK
        pltpu.sync_copy(idx2_hbm.at[pl.ds(rowblk, NCHUNK)], idx_v)

        @pl.when(cid == NC - 1)
        def _():
            pltpu.sync_copy(zeros_hbm, stage_v)
            for k in range(hw // (NS * 128)):
                pltpu.sync_copy(
                    stage_v,
                    attsh.at[pl.ds(sid * (hw // NS) + k * 128, 128)])

        plsc.subcore_barrier()

        @pl.when(cid == NC - 1)
        def _():
            pltpu.sync_copy(ones_hbm, stage_v)
            for j in range(NCHUNK):
                pltpu.sync_copy(stage_v, attsh.at[idx_v.at[j]], add=True)

        off = (base // hw) * hw
        for j in range(NCHUNK):
            for i in range(128 // L):
                sl = pl.ds(i * L, L)
                idx_v[j, sl] = idx_v[j, sl] + off
        copies = [
            pltpu.async_copy(fsn_hbm.at[idx_v.at[j]],
                             rows_v.at[pl.ds(j * 128, 128)], sem)
            for j in range(NCHUNK)
        ]
        wbs = []
        for j, cp in enumerate(copies):
            cp.wait()
            wbs.append(
                pltpu.async_copy(rows_v.at[pl.ds(j * 128, 128)],
                                 sel_hbm.at[pl.ds(base + j * 128, 128)],
                                 sem2))
        for wb in wbs:
            wb.wait()

        plsc.subcore_barrier()

        @pl.when((sid == 0) & (cid == NC - 1))
        def _():
            pltpu.sync_copy(attsh, att_hbm)

    ones = jnp.ones((128, 16), jnp.float32)
    zeros = jnp.zeros((128, 16), jnp.float32)
    return sc_kernel(fsn_flat, idx2, ones, zeros)


def _pass_c_body(C, qn_ref, sel_ref, w_ref, mcol_ref, xT_ref, fore_ref,
                 wct_ref, bc_ref, attp_ref, out_ref, att_ref):
    att_ref[...] = jnp.minimum(attp_ref[:, 0:1], 1.0)
    w = w_ref[0]
    mx = jnp.max(w)
    e = jnp.exp(w - mx)
    sm = e / jnp.sum(e)

    sel = sel_ref[0]
    qn = qn_ref[0]
    hyb = (jnp.dot(sel, wct_ref[:C, :], preferred_element_type=jnp.float32)
           * sm
           + jnp.dot(qn, wct_ref[C:, :], preferred_element_type=jnp.float32)
           + bc_ref[...])
    vm = jnp.where(fore_ref[0] > 0.5, mcol_ref[0, 0:1, :], 0.0)
    refined = hyb * vm + qn * (1.0 - vm)
    mc = mcol_ref[0]
    out_ref[0] = refined * mc + xT_ref[0] * (1.0 - mc)


def _run_pass_c(qnT, selT, w, mcol, xT, fore, wcT, bc2, attp):
    B, HW, C = qnT.shape
    f32 = jnp.float32
    return pl.pallas_call(
        functools.partial(_pass_c_body, C),
        grid=(B,),
        in_specs=[
            pl.BlockSpec((1, HW, C), lambda b: (b, 0, 0)),
            pl.BlockSpec((1, HW, C), lambda b: (b, 0, 0)),
            pl.BlockSpec((1, HW, 1), lambda b: (b, 0, 0)),
            pl.BlockSpec((1, HW, 1), lambda b: (b, 0, 0)),
            pl.BlockSpec((1, HW, C), lambda b: (b, 0, 0)),
            pl.BlockSpec((1, 1, 1), lambda b: (b, 0, 0)),
            pl.BlockSpec((2 * C, C), lambda b: (0, 0)),
            pl.BlockSpec((1, C), lambda b: (0, 0)),
            pl.BlockSpec((HW, 16), lambda b: (0, 0)),
        ],
        out_specs=[
            pl.BlockSpec((1, HW, C), lambda b: (b, 0, 0)),
            pl.BlockSpec((HW, 1), lambda b: (0, 0)),
        ],
        out_shape=[
            jax.ShapeDtypeStruct((B, HW, C), f32),
            jax.ShapeDtypeStruct((HW, 1), f32),
        ],
    )(qnT, selT, w, mcol, xT, fore, wcT, bc2, attp)


def kernel(x, mask, Wc, bc):
    B, C, H, Wd = x.shape
    HW = H * Wd
    xr = x.reshape(B, C, HW)
    xT = xr.transpose(0, 2, 1)
    mflat = mask.reshape(B, HW)
    mrow = mflat.reshape(B, 1, HW)
    mcol = mflat.reshape(B, HW, 1)

    qnT, fsnT, w, idx, fore = _run_pass_a(xT, xr, mrow, mcol)

    idx2 = idx.reshape(B * HW // 128, 128)
    fsn_flat = fsnT.reshape(B * HW, C)
    sel_flat, attp = _run_pass_b(fsn_flat, idx2, HW)
    selT = sel_flat.reshape(B, HW, C)

    outT, attv = _run_pass_c(qnT, selT, w, mcol, xT, fore,
                             Wc.T.reshape(2 * C, C), bc.reshape(1, C), attp)
    out = outT.transpose(0, 2, 1).reshape(B, C, H, Wd)

    att = jnp.broadcast_to(attv.reshape(1, HW), (B, HW)).reshape(B, 1, H, Wd)
    att = jnp.repeat(jnp.repeat(att, 8, axis=2), 8, axis=3)
    return out, att


# speedup vs baseline: 1.1960x; 1.0984x over previous
import functools

import jax
import jax.numpy as jnp
from jax import lax
from jax.experimental import pallas as pl
from jax.experimental.pallas import tpu as pltpu
from jax.experimental.pallas import tpu_sc as plsc

_TR = 1024


def _pass_a_body(hw, nb, x_ref, mrow_ref, mcol_ref,
                 qn_ref, fsn_ref, w_ref, idx_ref, fore_ref,
                 fs_cn_ref):
    t = pl.program_id(1)

    @pl.when(t == 0)
    def _():
        xb = x_ref[0]
        mr = mrow_ref[0]
        fs = xb * (1.0 - mr)
        nrm = jnp.sqrt(jnp.sum(fs * fs, axis=0, keepdims=True)) + 1e-8
        fs_cn_ref[...] = fs / nrm

    xt = x_ref[0, :, pl.ds(t * _TR, _TR)].T
    mc = mcol_ref[0]

    q = xt * mc
    qn = q / (jnp.sqrt(jnp.sum(q * q, axis=1, keepdims=True)) + 1e-8)
    qn_ref[0] = qn

    s_rows = xt * (1.0 - mc)
    sn = s_rows / (jnp.sqrt(jnp.sum(s_rows * s_rows, axis=1, keepdims=True))
                   + 1e-8)
    fsn_ref[0] = sn

    simi = jnp.dot(qn, fs_cn_ref[...], preferred_element_type=jnp.float32)
    w = jnp.max(simi, axis=1, keepdims=True)
    w_ref[0] = w
    iif = lax.broadcasted_iota(jnp.int32, simi.shape, 1).astype(jnp.float32)
    idxf = jnp.min(jnp.where(simi == w, iif, float(hw)), axis=1,
                   keepdims=True)
    idx = idxf.astype(jnp.int32)
    idx_ref[0] = idx

    c0 = jnp.max(simi[:, 0:1], axis=0, keepdims=True)

    @pl.when(t == 0)
    def _():
        fore_ref[0] = c0

    @pl.when(t != 0)
    def _():
        fore_ref[0] = jnp.maximum(fore_ref[0], c0)


def _run_pass_a(xr, mrow, mcol):
    B, C, HW = xr.shape
    T = HW // _TR
    f32 = jnp.float32
    return pl.pallas_call(
        functools.partial(_pass_a_body, HW, B),
        grid=(B, T),
        in_specs=[
            pl.BlockSpec((1, C, HW), lambda b, t: (b, 0, 0)),
            pl.BlockSpec((1, 1, HW), lambda b, t: (b, 0, 0)),
            pl.BlockSpec((1, _TR, 1), lambda b, t: (b, t, 0)),
        ],
        out_specs=[
            pl.BlockSpec((1, _TR, C), lambda b, t: (b, t, 0)),
            pl.BlockSpec((1, _TR, C), lambda b, t: (b, t, 0)),
            pl.BlockSpec((1, _TR, 1), lambda b, t: (b, t, 0)),
            pl.BlockSpec((1, _TR, 1), lambda b, t: (b, t, 0)),
            pl.BlockSpec((1, 1, 1), lambda b, t: (b, 0, 0)),
        ],
        out_shape=[
            jax.ShapeDtypeStruct((B, HW, C), f32),
            jax.ShapeDtypeStruct((B, HW, C), f32),
            jax.ShapeDtypeStruct((B, HW, 1), f32),
            jax.ShapeDtypeStruct((B, HW, 1), jnp.int32),
            jax.ShapeDtypeStruct((B, 1, 1), f32),
        ],
        scratch_shapes=[pltpu.VMEM((C, HW), f32)],
    )(xr, mrow, mcol)


def _run_pass_b(fsn_flat, idx2, hw):
    ROWS, C = fsn_flat.shape
    info = plsc.get_sparse_core_info()
    NC, NS, L = info.num_cores, info.num_subcores, info.num_lanes
    NW = NC * NS
    RPW = ROWS // NW
    NCHUNK = RPW // 128
    f32 = jnp.float32

    mesh = plsc.VectorSubcoreMesh(core_axis_name="c", subcore_axis_name="s")

    @functools.partial(
        pl.kernel,
        out_type=[
            jax.ShapeDtypeStruct((ROWS, C), f32),
            jax.ShapeDtypeStruct((hw, 16), f32),
        ],
        mesh=mesh,
        scratch_types=[
            pltpu.VMEM((NCHUNK, 128), jnp.int32),
            pltpu.VMEM((RPW, C), f32),
            pltpu.SemaphoreType.DMA,
            pltpu.SemaphoreType.DMA,
            pltpu.VMEM((128, 16), f32),
            pltpu.VMEM_SHARED((hw, 16), f32),
        ],
        compiler_params=pltpu.CompilerParams(use_tc_tiling_on_sc=False),
    )
    def sc_kernel(fsn_hbm, idx2_hbm, ones_hbm, zeros_hbm,
                  sel_hbm, att_hbm,
                  idx_v, rows_v, sem, sem2, stage_v, attsh):
        cid = lax.axis_index("c")
        sid = lax.axis_index("s")
        wid = cid * NS + sid
        base = wid * RPW
        rowblk = wid * NCHUNK
        pltpu.sync_copy(idx2_hbm.at[pl.ds(rowblk, NCHUNK)], idx_v)

        @pl.when(cid == NC - 1)
        def _():
            pltpu.sync_copy(zeros_hbm, stage_v)
            for k in range(hw // (NS * 128)):
                pltpu.sync_copy(
                    stage_v,
                    attsh.at[pl.ds(sid * (hw // NS) + k * 128, 128)])

        plsc.subcore_barrier()

        @pl.when(cid == NC - 1)
        def _():
            pltpu.sync_copy(ones_hbm, stage_v)
            for j in range(NCHUNK):
                pltpu.sync_copy(stage_v, attsh.at[idx_v.at[j]], add=True)

        off = (base // hw) * hw
        for j in range(NCHUNK):
            for i in range(128 // L):
                sl = pl.ds(i * L, L)
                idx_v[j, sl] = idx_v[j, sl] + off
        copies = [
            pltpu.async_copy(fsn_hbm.at[idx_v.at[j]],
                             rows_v.at[pl.ds(j * 128, 128)], sem)
            for j in range(NCHUNK)
        ]
        wbs = []
        for j, cp in enumerate(copies):
            cp.wait()
            wbs.append(
                pltpu.async_copy(rows_v.at[pl.ds(j * 128, 128)],
                                 sel_hbm.at[pl.ds(base + j * 128, 128)],
                                 sem2))
        for wb in wbs:
            wb.wait()

        plsc.subcore_barrier()

        @pl.when((sid == 0) & (cid == NC - 1))
        def _():
            pltpu.sync_copy(attsh, att_hbm)

    ones = jnp.ones((128, 16), jnp.float32)
    zeros = jnp.zeros((128, 16), jnp.float32)
    return sc_kernel(fsn_flat, idx2, ones, zeros)


def _pass_c_body(C, qn_ref, sel_ref, w_ref, mrow_ref, x_ref, fore_ref,
                 wct_ref, bc_ref, attp_ref, out_ref, att_ref):
    att_ref[...] = jnp.minimum(attp_ref[:, 0:1], 1.0)
    w = w_ref[0]
    mx = jnp.max(w)
    e = jnp.exp(w - mx)
    sm = e / jnp.sum(e)

    sel = sel_ref[0]
    qn = qn_ref[0]
    hyb = (jnp.dot(sel, wct_ref[:C, :], preferred_element_type=jnp.float32)
           * sm
           + jnp.dot(qn, wct_ref[C:, :], preferred_element_type=jnp.float32)
           + bc_ref[...])
    vm = jnp.where(fore_ref[0] > 0.5, mrow_ref[0, :, 0:1], 0.0)
    refined = hyb * vm + qn * (1.0 - vm)
    mr = mrow_ref[0]
    out_ref[0] = refined.T * mr + x_ref[0] * (1.0 - mr)


def _run_pass_c(qnT, selT, w, mrow, xr, fore, wcT, bc2, attp):
    B, C, HW = xr.shape
    f32 = jnp.float32
    return pl.pallas_call(
        functools.partial(_pass_c_body, C),
        grid=(B,),
        in_specs=[
            pl.BlockSpec((1, HW, C), lambda b: (b, 0, 0)),
            pl.BlockSpec((1, HW, C), lambda b: (b, 0, 0)),
            pl.BlockSpec((1, HW, 1), lambda b: (b, 0, 0)),
            pl.BlockSpec((1, 1, HW), lambda b: (b, 0, 0)),
            pl.BlockSpec((1, C, HW), lambda b: (b, 0, 0)),
            pl.BlockSpec((1, 1, 1), lambda b: (b, 0, 0)),
            pl.BlockSpec((2 * C, C), lambda b: (0, 0)),
            pl.BlockSpec((1, C), lambda b: (0, 0)),
            pl.BlockSpec((HW, 16), lambda b: (0, 0)),
        ],
        out_specs=[
            pl.BlockSpec((1, C, HW), lambda b: (b, 0, 0)),
            pl.BlockSpec((HW, 1), lambda b: (0, 0)),
        ],
        out_shape=[
            jax.ShapeDtypeStruct((B, C, HW), f32),
            jax.ShapeDtypeStruct((HW, 1), f32),
        ],
    )(qnT, selT, w, mrow, xr, fore, wcT, bc2, attp)


def kernel(x, mask, Wc, bc):
    B, C, H, Wd = x.shape
    HW = H * Wd
    xr = x.reshape(B, C, HW)
    mflat = mask.reshape(B, HW)
    mrow = mflat.reshape(B, 1, HW)
    mcol = mflat.reshape(B, HW, 1)

    qnT, fsnT, w, idx, fore = _run_pass_a(xr, mrow, mcol)

    idx2 = idx.reshape(B * HW // 128, 128)
    fsn_flat = fsnT.reshape(B * HW, C)
    sel_flat, attp = _run_pass_b(fsn_flat, idx2, HW)
    selT = sel_flat.reshape(B, HW, C)

    out_cn, attv = _run_pass_c(qnT, selT, w, mrow, xr, fore,
                               Wc.T.reshape(2 * C, C), bc.reshape(1, C),
                               attp)
    out = out_cn.reshape(B, C, H, Wd)

    att = jnp.broadcast_to(attv.reshape(1, HW), (B, HW)).reshape(B, 1, H, Wd)
    att = jnp.repeat(jnp.repeat(att, 8, axis=2), 8, axis=3)
    return out, att
